# baseline (speedup 1.0000x reference)
import functools

import jax
import jax.numpy as jnp
from jax import lax
from jax.experimental import pallas as pl
from jax.experimental.pallas import tpu as pltpu
from jax.experimental.pallas import tpu_sc as plsc

N_EXP = 8
D = 1024
DFF = 4096
N_TOK = 2048
TB = 256
FB = 2048
NB = 24
P = NB * TB
LB_COEF = 0.01
NC = 2
NS = 16
NW = NC * NS
TPW = N_TOK // NW


def _router_body(x_ref, wg_ref, easgn_ref, wasgn_ref, counts_ref, aux_ref):
    x = x_ref[...]
    wg = wg_ref[...]
    logits = lax.dot_general(
        wg, x, (((1,), (1,)), ((), ())),
        preferred_element_type=jnp.float32)
    m = jnp.max(logits, axis=0, keepdims=True)
    ex = jnp.exp(logits - m)
    probs = ex / jnp.sum(ex, axis=0, keepdims=True)

    eidx = lax.broadcasted_iota(jnp.int32, (N_EXP, N_TOK), 0)
    m1 = jnp.max(probs, axis=0, keepdims=True)
    am1 = jnp.min(jnp.where(probs == m1, eidx, N_EXP), axis=0, keepdims=True)
    p2 = jnp.where(eidx == am1, -1.0, probs)
    m2 = jnp.max(p2, axis=0, keepdims=True)
    am2 = jnp.min(jnp.where(p2 == m2, eidx, N_EXP), axis=0, keepdims=True)

    denom = m1 + m2
    easgn_ref[...] = jnp.concatenate([am1, am2], axis=0)
    wasgn_ref[...] = jnp.concatenate([m1 / denom, m2 / denom], axis=0)

    oh1 = (eidx == am1).astype(jnp.float32)
    oh2 = (eidx == am2).astype(jnp.float32)
    sel = (lax.broadcasted_iota(jnp.int32, (NW, N_TOK), 0)
           == lax.broadcasted_iota(jnp.int32, (NW, N_TOK), 1) // TPW
           ).astype(jnp.float32)
    cnts = lax.dot_general(
        sel, oh1 + oh2, (((1,), (1,)), ((), ())),
        preferred_element_type=jnp.float32)
    counts_ref[...] = jnp.concatenate(
        [cnts, jnp.zeros((NW, 16 - N_EXP), jnp.float32)], axis=1
    ).astype(jnp.int32)

    f_i = jnp.sum(oh1, axis=1, keepdims=True)
    p_i = jnp.sum(probs, axis=1, keepdims=True)
    aux = (LB_COEF / (N_TOK * N_TOK)) * jnp.sum(f_i * p_i, axis=0,
                                                keepdims=True)
    aux_ref[...] = aux


def _dispatch_body(x_hbm, easgn_hbm, counts_hbm, xd_hbm, pos_hbm, bexp_hbm,
                   allcnt_v, e2d_v, dst_v, xrows_v, bexp_v, sem):
    wid = lax.axis_index("s") * NC + lax.axis_index("c")
    base = wid * TPW
    lanes = lax.iota(jnp.int32, 16)
    zeros = jnp.zeros((16,), jnp.int32)
    widv = jnp.full((16,), wid, jnp.int32)

    pltpu.sync_copy(counts_hbm, allcnt_v)
    pltpu.sync_copy(easgn_hbm.at[0, pl.ds(base, TPW)], e2d_v.at[0])
    pltpu.sync_copy(easgn_hbm.at[1, pl.ds(base, TPW)], e2d_v.at[1])

    totals = zeros
    prefix = zeros
    for t in range(NW):
        v = allcnt_v[t, pl.ds(0, 16)]
        totals = totals + v
        tv = jnp.full((16,), t, jnp.int32)
        prefix = prefix + jnp.where(tv < widv, v, zeros)

    pc = lax.shift_left(lax.shift_right_logical(totals + (TB - 1), 8), 8)
    pad_off = plsc.cumsum(pc) - pc
    mybase = pad_off + prefix

    running = [jnp.sum(jnp.where(lanes == e, mybase, zeros))
               for e in range(N_EXP)]
    for k in range(2):
        for j in range(TPW // 16):
            ev = e2d_v[k, pl.ds(j * 16, 16)]
            dstv = zeros
            for e in range(N_EXP):
                msk = ev == e
                mi = msk.astype(jnp.int32)
                cs = plsc.cumsum(mi)
                dstv = jnp.where(msk, running[e] + cs - 1, dstv)
                running[e] = running[e] + jnp.sum(mi)
            dst_v[k, pl.ds(j * 16, 16)] = dstv

    pltpu.sync_copy(dst_v.at[0], pos_hbm.at[0, pl.ds(base, TPW)])
    pltpu.sync_copy(dst_v.at[1], pos_hbm.at[1, pl.ds(base, TPW)])

    pltpu.sync_copy(x_hbm.at[pl.ds(base, TPW)], xrows_v)
    pltpu.async_copy(xrows_v, xd_hbm.at[dst_v.at[0]], sem).wait()
    pltpu.async_copy(xrows_v, xd_hbm.at[dst_v.at[1]], sem).wait()

    @pl.when(wid == 0)
    def _bexp():
        bstart = lax.shift_right_logical(pad_off, 8)
        nb = lax.shift_right_logical(pc, 8)
        nblk = jnp.sum(nb)
        elast = jnp.max(jnp.where(totals > 0, lanes, zeros))
        bs = [jnp.sum(jnp.where(lanes == e, bstart, zeros))
              for e in range(N_EXP)]
        nbs = [jnp.sum(jnp.where(lanes == e, nb, zeros))
               for e in range(N_EXP)]
        for v in range(2):
            bi = lanes + v * 16
            be = jnp.full((16,), 0, jnp.int32) + elast
            for e in range(N_EXP):
                msk = jnp.logical_and(bi >= bs[e], bi < bs[e] + nbs[e])
                be = jnp.where(msk, e, be)
            be = jnp.where(bi == NB, nblk, be)
            bexp_v[pl.ds(v * 16, 16)] = be
        pltpu.sync_copy(bexp_v, bexp_hbm)


def _grouped_body(s_ref, xd_ref, w1_ref, b1_ref, w2_ref, b2_ref, yd_ref):
    f = pl.program_id(0)
    b = pl.program_id(1)
    e = s_ref[b]
    nblk = s_ref[NB]

    @pl.when(b < nblk)
    def _compute():
        x = xd_ref[...]
        w1 = w1_ref[0]
        b1_blk = b1_ref[pl.ds(e, 1), pl.ds(pl.multiple_of(f * FB, FB), FB)]
        h = lax.dot_general(
            x, w1, (((1,), (1,)), ((), ())),
            preferred_element_type=jnp.float32) + b1_blk
        g = 0.5 * h * (1.0 + lax.erf(h * 0.7071067811865476))
        w2 = w2_ref[0]
        part = lax.dot_general(
            g, w2, (((1,), (1,)), ((), ())),
            preferred_element_type=jnp.float32)
        b2_term = jnp.where(f == 0, b2_ref[pl.ds(e, 1), :],
                            jnp.zeros((1, D), jnp.float32))
        yd_ref[0] = part + b2_term


def _combine_body(yd_hbm, pos_hbm, wasgn_hbm, out_hbm,
                  p1_v, p2_v, q1_v, q2_v, w1_v, w2_v,
                  r1a_v, r1b_v, r2a_v, r2b_v, ob_v, sem):
    wid = lax.axis_index("s") * NC + lax.axis_index("c")
    base = wid * TPW
    QB = 16

    pltpu.sync_copy(wasgn_hbm.at[0, pl.ds(base, TPW)], w1_v)
    pltpu.sync_copy(wasgn_hbm.at[1, pl.ds(base, TPW)], w2_v)

    for q in range(TPW // QB):
        pltpu.sync_copy(pos_hbm.at[0, pl.ds(base + q * QB, QB)], p1_v.at[q])
        pltpu.sync_copy(pos_hbm.at[1, pl.ds(base + q * QB, QB)], p2_v.at[q])
        q1_v[q, pl.ds(0, QB)] = p1_v[q, pl.ds(0, QB)] + P
        q2_v[q, pl.ds(0, QB)] = p2_v[q, pl.ds(0, QB)] + P
        c1 = pltpu.async_copy(yd_hbm.at[p1_v.at[q]], r1a_v, sem)
        c2 = pltpu.async_copy(yd_hbm.at[q1_v.at[q]], r1b_v, sem)
        c3 = pltpu.async_copy(yd_hbm.at[p2_v.at[q]], r2a_v, sem)
        c4 = pltpu.async_copy(yd_hbm.at[q2_v.at[q]], r2b_v, sem)
        c1.wait(); c2.wait(); c3.wait(); c4.wait()

        def _row(r, carry):
            idx = jnp.full((16,), q * QB + r, jnp.int32)
            w1s = plsc.load_gather(w1_v, [idx])
            w2s = plsc.load_gather(w2_v, [idx])

            def _col(c, carry2):
                a = r1a_v[r, pl.ds(c * 16, 16)] + r1b_v[r, pl.ds(c * 16, 16)]
                bb = r2a_v[r, pl.ds(c * 16, 16)] + r2b_v[r, pl.ds(c * 16, 16)]
                ob_v[r, pl.ds(c * 16, 16)] = w1s * a + w2s * bb
                return carry2

            lax.fori_loop(0, D // 16, _col, 0)
            return carry

        lax.fori_loop(0, QB, _row, 0)
        pltpu.sync_copy(ob_v, out_hbm.at[pl.ds(base + q * QB, QB)])


_SC_MESH = plsc.VectorSubcoreMesh(core_axis_name="c", subcore_axis_name="s",
                                  num_cores=NC, num_subcores=NS)

_dispatch = pl.kernel(
    _dispatch_body,
    out_type=(
        jax.ShapeDtypeStruct((P, D), jnp.float32),
        jax.ShapeDtypeStruct((2, N_TOK), jnp.int32),
        jax.ShapeDtypeStruct((NB + 8, ), jnp.int32),
    ),
    mesh=_SC_MESH,
    compiler_params=pltpu.CompilerParams(needs_layout_passes=False),
    scratch_types=[
        pltpu.VMEM((NW, 16), jnp.int32),
        pltpu.VMEM((2, TPW), jnp.int32),
        pltpu.VMEM((2, TPW), jnp.int32),
        pltpu.VMEM((TPW, D), jnp.float32),
        pltpu.VMEM((NB + 8,), jnp.int32),
        pltpu.SemaphoreType.DMA,
    ],
)

_combine = pl.kernel(
    _combine_body,
    out_type=jax.ShapeDtypeStruct((N_TOK, D), jnp.float32),
    mesh=_SC_MESH,
    compiler_params=pltpu.CompilerParams(needs_layout_passes=False),
    scratch_types=[
        pltpu.VMEM((4, 16), jnp.int32),
        pltpu.VMEM((4, 16), jnp.int32),
        pltpu.VMEM((4, 16), jnp.int32),
        pltpu.VMEM((4, 16), jnp.int32),
        pltpu.VMEM((TPW,), jnp.float32),
        pltpu.VMEM((TPW,), jnp.float32),
        pltpu.VMEM((16, D), jnp.float32),
        pltpu.VMEM((16, D), jnp.float32),
        pltpu.VMEM((16, D), jnp.float32),
        pltpu.VMEM((16, D), jnp.float32),
        pltpu.VMEM((16, D), jnp.float32),
        pltpu.SemaphoreType.DMA,
    ],
)


@jax.jit
def kernel(x, W_gate, W1, b1, W2, b2):
    x_flat = x.reshape(N_TOK, D)

    easgn, wasgn, counts, aux = pl.pallas_call(
        _router_body,
        out_shape=(
            jax.ShapeDtypeStruct((2, N_TOK), jnp.int32),
            jax.ShapeDtypeStruct((2, N_TOK), jnp.float32),
            jax.ShapeDtypeStruct((NW, 16), jnp.int32),
            jax.ShapeDtypeStruct((1, 1), jnp.float32),
        ),
    )(x_flat, W_gate)

    xd, pos, bexp = _dispatch(x_flat, easgn, counts)

    yd = pl.pallas_call(
        _grouped_body,
        grid_spec=pltpu.PrefetchScalarGridSpec(
            num_scalar_prefetch=1,
            grid=(DFF // FB, NB),
            in_specs=[
                pl.BlockSpec((TB, D), lambda f, b, s: (b, 0)),
                pl.BlockSpec((1, FB, D), lambda f, b, s: (s[b], f, 0)),
                pl.BlockSpec((N_EXP, DFF), lambda f, b, s: (0, 0)),
                pl.BlockSpec((1, D, FB), lambda f, b, s: (s[b], 0, f)),
                pl.BlockSpec((N_EXP, D), lambda f, b, s: (0, 0)),
            ],
            out_specs=pl.BlockSpec((1, TB, D), lambda f, b, s: (f, b, 0)),
        ),
        out_shape=jax.ShapeDtypeStruct((DFF // FB, P, D), jnp.float32),
        compiler_params=pltpu.CompilerParams(
            dimension_semantics=("arbitrary", "arbitrary"),
            vmem_limit_bytes=56 * 1024 * 1024,
        ),
    )(bexp, xd, W1, b1, W2, b2)

    out = _combine(yd.reshape((DFF // FB) * P, D), pos, wasgn)
    return out.reshape(1, N_TOK, D), aux.reshape(())


# speedup vs baseline: 1.1494x; 1.1494x over previous
import functools

import jax
import jax.numpy as jnp
from jax import lax
from jax.experimental import pallas as pl
from jax.experimental.pallas import tpu as pltpu
from jax.experimental.pallas import tpu_sc as plsc

N_EXP = 8
D = 1024
DFF = 4096
N_TOK = 2048
TB = 256
FB = 2048
NB = 24
P = NB * TB
LB_COEF = 0.01
NC = 2
NS = 16
NW = NC * NS
TPW = N_TOK // NW


def _router_body(x_ref, wg_ref, easgn_ref, wasgn_ref, counts_ref, aux_ref):
    x = x_ref[...]
    wg = wg_ref[...]
    logits = lax.dot_general(
        wg, x, (((1,), (1,)), ((), ())),
        preferred_element_type=jnp.float32)
    m = jnp.max(logits, axis=0, keepdims=True)
    ex = jnp.exp(logits - m)
    probs = ex / jnp.sum(ex, axis=0, keepdims=True)

    eidx = lax.broadcasted_iota(jnp.int32, (N_EXP, N_TOK), 0)
    m1 = jnp.max(probs, axis=0, keepdims=True)
    am1 = jnp.min(jnp.where(probs == m1, eidx, N_EXP), axis=0, keepdims=True)
    p2 = jnp.where(eidx == am1, -1.0, probs)
    m2 = jnp.max(p2, axis=0, keepdims=True)
    am2 = jnp.min(jnp.where(p2 == m2, eidx, N_EXP), axis=0, keepdims=True)

    denom = m1 + m2
    easgn_ref[...] = jnp.concatenate([am1, am2], axis=0)
    wasgn_ref[...] = jnp.concatenate([m1 / denom, m2 / denom], axis=0)

    oh1 = (eidx == am1).astype(jnp.float32)
    oh2 = (eidx == am2).astype(jnp.float32)
    sel = (lax.broadcasted_iota(jnp.int32, (NW, N_TOK), 0)
           == lax.broadcasted_iota(jnp.int32, (NW, N_TOK), 1) // TPW
           ).astype(jnp.float32)
    cnts = lax.dot_general(
        sel, oh1 + oh2, (((1,), (1,)), ((), ())),
        preferred_element_type=jnp.float32)
    counts_ref[...] = jnp.concatenate(
        [cnts, jnp.zeros((NW, 16 - N_EXP), jnp.float32)], axis=1
    ).astype(jnp.int32)

    f_i = jnp.sum(oh1, axis=1, keepdims=True)
    p_i = jnp.sum(probs, axis=1, keepdims=True)
    aux = (LB_COEF / (N_TOK * N_TOK)) * jnp.sum(f_i * p_i, axis=0,
                                                keepdims=True)
    aux_ref[...] = aux


def _dispatch_body(x_hbm, easgn_hbm, counts_hbm, xd_hbm, pos_hbm, bexp_hbm,
                   allcnt_v, e2d_v, dst_v, xrows_v, bexp_v, sem):
    wid = lax.axis_index("s") * NC + lax.axis_index("c")
    base = wid * TPW
    lanes = lax.iota(jnp.int32, 16)
    zeros = jnp.zeros((16,), jnp.int32)
    widv = jnp.full((16,), wid, jnp.int32)

    pltpu.sync_copy(counts_hbm, allcnt_v)
    pltpu.sync_copy(easgn_hbm.at[0, pl.ds(base, TPW)], e2d_v.at[0])
    pltpu.sync_copy(easgn_hbm.at[1, pl.ds(base, TPW)], e2d_v.at[1])

    totals = zeros
    prefix = zeros
    for t in range(NW):
        v = allcnt_v[t, pl.ds(0, 16)]
        totals = totals + v
        tv = jnp.full((16,), t, jnp.int32)
        prefix = prefix + jnp.where(tv < widv, v, zeros)

    pc = lax.shift_left(lax.shift_right_logical(totals + (TB - 1), 8), 8)
    pad_off = plsc.cumsum(pc) - pc
    mybase = pad_off + prefix

    running = [jnp.sum(jnp.where(lanes == e, mybase, zeros))
               for e in range(N_EXP)]
    for k in range(2):
        for j in range(TPW // 16):
            ev = e2d_v[k, pl.ds(j * 16, 16)]
            dstv = zeros
            for e in range(N_EXP):
                msk = ev == e
                mi = msk.astype(jnp.int32)
                cs = plsc.cumsum(mi)
                dstv = jnp.where(msk, running[e] + cs - 1, dstv)
                running[e] = running[e] + jnp.sum(mi)
            dst_v[k, pl.ds(j * 16, 16)] = dstv

    pltpu.sync_copy(dst_v.at[0], pos_hbm.at[0, pl.ds(base, TPW)])
    pltpu.sync_copy(dst_v.at[1], pos_hbm.at[1, pl.ds(base, TPW)])

    pltpu.sync_copy(x_hbm.at[pl.ds(base, TPW)], xrows_v)
    pltpu.async_copy(xrows_v, xd_hbm.at[dst_v.at[0]], sem).wait()
    pltpu.async_copy(xrows_v, xd_hbm.at[dst_v.at[1]], sem).wait()

    @pl.when(wid == 0)
    def _bexp():
        bstart = lax.shift_right_logical(pad_off, 8)
        nb = lax.shift_right_logical(pc, 8)
        nblk = jnp.sum(nb)
        elast = jnp.max(jnp.where(totals > 0, lanes, zeros))
        bs = [jnp.sum(jnp.where(lanes == e, bstart, zeros))
              for e in range(N_EXP)]
        nbs = [jnp.sum(jnp.where(lanes == e, nb, zeros))
               for e in range(N_EXP)]
        for v in range(2):
            bi = lanes + v * 16
            be = jnp.full((16,), 0, jnp.int32) + elast
            for e in range(N_EXP):
                msk = jnp.logical_and(bi >= bs[e], bi < bs[e] + nbs[e])
                be = jnp.where(msk, e, be)
            be = jnp.where(bi == NB, nblk, be)
            bexp_v[pl.ds(v * 16, 16)] = be
        pltpu.sync_copy(bexp_v, bexp_hbm)


def _gelu(h):
    return 0.5 * h * (1.0 + lax.erf(h * 0.7071067811865476))


def _grouped_body(s_ref, xd_ref, w1a_ref, w1b_ref, b1_ref, w2a_ref, w2b_ref,
                  b2_ref, yd_ref):
    b = pl.program_id(0)
    e = s_ref[b]
    nblk = s_ref[NB]

    @pl.when(b < nblk)
    def _compute():
        x = xd_ref[...]
        ha = lax.dot_general(
            x, w1a_ref[0], (((1,), (1,)), ((), ())),
            preferred_element_type=jnp.float32)
        hb = lax.dot_general(
            x, w1b_ref[0], (((1,), (1,)), ((), ())),
            preferred_element_type=jnp.float32)
        b1_row = b1_ref[pl.ds(e, 1), :]
        g = _gelu(jnp.concatenate([ha, hb], axis=1) + b1_row)
        pa = lax.dot_general(
            g, w2a_ref[0], (((1,), (1,)), ((), ())),
            preferred_element_type=jnp.float32)
        pb = lax.dot_general(
            g, w2b_ref[0], (((1,), (1,)), ((), ())),
            preferred_element_type=jnp.float32)
        yd_ref[...] = (jnp.concatenate([pa, pb], axis=1)
                       + b2_ref[pl.ds(e, 1), :])


def _combine_body(yd_hbm, pos_hbm, wasgn_hbm, out_hbm,
                  p1_v, p2_v, w1_v, w2_v, rows1_v, rows2_v, ob_v, sem):
    wid = lax.axis_index("s") * NC + lax.axis_index("c")
    base = wid * TPW
    half = TPW // 2

    pltpu.sync_copy(wasgn_hbm.at[0, pl.ds(base, TPW)], w1_v)
    pltpu.sync_copy(wasgn_hbm.at[1, pl.ds(base, TPW)], w2_v)

    for h in range(2):
        pltpu.sync_copy(pos_hbm.at[0, pl.ds(base + h * half, half)],
                        p1_v.at[h])
        pltpu.sync_copy(pos_hbm.at[1, pl.ds(base + h * half, half)],
                        p2_v.at[h])
        pltpu.async_copy(yd_hbm.at[p1_v.at[h]], rows1_v, sem).wait()
        pltpu.async_copy(yd_hbm.at[p2_v.at[h]], rows2_v, sem).wait()

        def _row(r, carry):
            idx = jnp.full((16,), h * half + r, jnp.int32)
            w1s = plsc.load_gather(w1_v, [idx])
            w2s = plsc.load_gather(w2_v, [idx])

            def _col(c, carry2):
                a = rows1_v[r, pl.ds(c * 16, 16)]
                bb = rows2_v[r, pl.ds(c * 16, 16)]
                ob_v[r, pl.ds(c * 16, 16)] = w1s * a + w2s * bb
                return carry2

            lax.fori_loop(0, D // 16, _col, 0)
            return carry

        lax.fori_loop(0, half, _row, 0)
        pltpu.sync_copy(ob_v, out_hbm.at[pl.ds(base + h * half, half)])


_SC_MESH = plsc.VectorSubcoreMesh(core_axis_name="c", subcore_axis_name="s",
                                  num_cores=NC, num_subcores=NS)

_dispatch = pl.kernel(
    _dispatch_body,
    out_type=(
        jax.ShapeDtypeStruct((P, D), jnp.float32),
        jax.ShapeDtypeStruct((2, N_TOK), jnp.int32),
        jax.ShapeDtypeStruct((NB + 8, ), jnp.int32),
    ),
    mesh=_SC_MESH,
    compiler_params=pltpu.CompilerParams(needs_layout_passes=False),
    scratch_types=[
        pltpu.VMEM((NW, 16), jnp.int32),
        pltpu.VMEM((2, TPW), jnp.int32),
        pltpu.VMEM((2, TPW), jnp.int32),
        pltpu.VMEM((TPW, D), jnp.float32),
        pltpu.VMEM((NB + 8,), jnp.int32),
        pltpu.SemaphoreType.DMA,
    ],
)

_combine = pl.kernel(
    _combine_body,
    out_type=jax.ShapeDtypeStruct((N_TOK, D), jnp.float32),
    mesh=_SC_MESH,
    compiler_params=pltpu.CompilerParams(needs_layout_passes=False),
    scratch_types=[
        pltpu.VMEM((2, TPW // 2), jnp.int32),
        pltpu.VMEM((2, TPW // 2), jnp.int32),
        pltpu.VMEM((TPW,), jnp.float32),
        pltpu.VMEM((TPW,), jnp.float32),
        pltpu.VMEM((TPW // 2, D), jnp.float32),
        pltpu.VMEM((TPW // 2, D), jnp.float32),
        pltpu.VMEM((TPW // 2, D), jnp.float32),
        pltpu.SemaphoreType.DMA,
    ],
)


@jax.jit
def kernel(x, W_gate, W1, b1, W2, b2):
    x_flat = x.reshape(N_TOK, D)

    easgn, wasgn, counts, aux = pl.pallas_call(
        _router_body,
        out_shape=(
            jax.ShapeDtypeStruct((2, N_TOK), jnp.int32),
            jax.ShapeDtypeStruct((2, N_TOK), jnp.float32),
            jax.ShapeDtypeStruct((NW, 16), jnp.int32),
            jax.ShapeDtypeStruct((1, 1), jnp.float32),
        ),
    )(x_flat, W_gate)

    xd, pos, bexp = _dispatch(x_flat, easgn, counts)

    yd = pl.pallas_call(
        _grouped_body,
        grid_spec=pltpu.PrefetchScalarGridSpec(
            num_scalar_prefetch=1,
            grid=(NB,),
            in_specs=[
                pl.BlockSpec((TB, D), lambda b, s: (b, 0)),
                pl.BlockSpec((1, DFF // 2, D), lambda b, s: (s[b], 0, 0)),
                pl.BlockSpec((1, DFF // 2, D), lambda b, s: (s[b], 1, 0)),
                pl.BlockSpec((N_EXP, DFF), lambda b, s: (0, 0)),
                pl.BlockSpec((1, D // 2, DFF), lambda b, s: (s[b], 0, 0),
                             pipeline_mode=pl.Buffered(buffer_count=1)),
                pl.BlockSpec((1, D // 2, DFF), lambda b, s: (s[b], 1, 0),
                             pipeline_mode=pl.Buffered(buffer_count=1)),
                pl.BlockSpec((N_EXP, D), lambda b, s: (0, 0)),
            ],
            out_specs=pl.BlockSpec((TB, D), lambda b, s: (b, 0)),
        ),
        out_shape=jax.ShapeDtypeStruct((P, D), jnp.float32),
        compiler_params=pltpu.CompilerParams(
            dimension_semantics=("arbitrary",),
            vmem_limit_bytes=60 * 1024 * 1024,
        ),
    )(bexp, xd, W1, W1, b1, W2, W2, b2)

    out = _combine(yd, pos, wasgn)
    return out.reshape(1, N_TOK, D), aux.reshape(())


# speedup vs baseline: 1.1766x; 1.0236x over previous
import functools

import jax
import jax.numpy as jnp
from jax import lax
from jax.experimental import pallas as pl
from jax.experimental.pallas import tpu as pltpu
from jax.experimental.pallas import tpu_sc as plsc

N_EXP = 8
D = 1024
DFF = 4096
N_TOK = 2048
TB = 256
FB = 2048
NB = 24
P = NB * TB
LB_COEF = 0.01
NC = 2
NS = 16
NW = NC * NS
TPW = N_TOK // NW


def _router_body(x_ref, wg_ref, easgn_ref, wasgn_ref, counts_ref, aux_ref):
    x = x_ref[...]
    wg = wg_ref[...]
    logits = lax.dot_general(
        wg, x, (((1,), (1,)), ((), ())),
        preferred_element_type=jnp.float32)
    m = jnp.max(logits, axis=0, keepdims=True)
    ex = jnp.exp(logits - m)
    probs = ex / jnp.sum(ex, axis=0, keepdims=True)

    eidx = lax.broadcasted_iota(jnp.int32, (N_EXP, N_TOK), 0)
    m1 = jnp.max(probs, axis=0, keepdims=True)
    am1 = jnp.min(jnp.where(probs == m1, eidx, N_EXP), axis=0, keepdims=True)
    p2 = jnp.where(eidx == am1, -1.0, probs)
    m2 = jnp.max(p2, axis=0, keepdims=True)
    am2 = jnp.min(jnp.where(p2 == m2, eidx, N_EXP), axis=0, keepdims=True)

    denom = m1 + m2
    easgn_ref[...] = jnp.concatenate([am1, am2], axis=0)
    wasgn_ref[...] = jnp.concatenate([m1 / denom, m2 / denom], axis=0)

    oh1 = (eidx == am1).astype(jnp.float32)
    oh2 = (eidx == am2).astype(jnp.float32)
    sel = (lax.broadcasted_iota(jnp.int32, (NW, N_TOK), 0)
           == lax.broadcasted_iota(jnp.int32, (NW, N_TOK), 1) // TPW
           ).astype(jnp.float32)
    cnts = lax.dot_general(
        sel, oh1 + oh2, (((1,), (1,)), ((), ())),
        preferred_element_type=jnp.float32)
    counts_ref[...] = jnp.concatenate(
        [cnts, jnp.zeros((NW, 16 - N_EXP), jnp.float32)], axis=1
    ).astype(jnp.int32)

    f_i = jnp.sum(oh1, axis=1, keepdims=True)
    p_i = jnp.sum(probs, axis=1, keepdims=True)
    aux = (LB_COEF / (N_TOK * N_TOK)) * jnp.sum(f_i * p_i, axis=0,
                                                keepdims=True)
    aux_ref[...] = aux


def _dispatch_body(x_hbm, easgn_hbm, counts_hbm, xd_hbm, pos_hbm, bexp_hbm,
                   allcnt_v, e2d_v, dst_v, xrows_v, bexp_v, sem):
    wid = lax.axis_index("s") * NC + lax.axis_index("c")
    base = wid * TPW
    lanes = lax.iota(jnp.int32, 16)
    zeros = jnp.zeros((16,), jnp.int32)
    widv = jnp.full((16,), wid, jnp.int32)

    xload = pltpu.async_copy(x_hbm.at[pl.ds(base, TPW)], xrows_v, sem)
    pltpu.sync_copy(counts_hbm, allcnt_v)
    pltpu.sync_copy(easgn_hbm.at[0, pl.ds(base, TPW)], e2d_v.at[0])
    pltpu.sync_copy(easgn_hbm.at[1, pl.ds(base, TPW)], e2d_v.at[1])

    totals = zeros
    prefix = zeros
    for t in range(NW):
        v = allcnt_v[t, pl.ds(0, 16)]
        totals = totals + v
        tv = jnp.full((16,), t, jnp.int32)
        prefix = prefix + jnp.where(tv < widv, v, zeros)

    pc = lax.shift_left(lax.shift_right_logical(totals + (TB - 1), 8), 8)
    pad_off = plsc.cumsum(pc) - pc
    mybase = pad_off + prefix

    running = [jnp.sum(jnp.where(lanes == e, mybase, zeros))
               for e in range(N_EXP)]
    for k in range(2):
        for j in range(TPW // 16):
            ev = e2d_v[k, pl.ds(j * 16, 16)]
            dstv = zeros
            for e in range(N_EXP):
                msk = ev == e
                mi = msk.astype(jnp.int32)
                cs = plsc.cumsum(mi)
                dstv = jnp.where(msk, running[e] + cs - 1, dstv)
                running[e] = running[e] + jnp.sum(mi)
            dst_v[k, pl.ds(j * 16, 16)] = dstv

    pltpu.sync_copy(dst_v.at[0], pos_hbm.at[0, pl.ds(base, TPW)])
    pltpu.sync_copy(dst_v.at[1], pos_hbm.at[1, pl.ds(base, TPW)])

    xload.wait()
    c1 = pltpu.async_copy(xrows_v, xd_hbm.at[dst_v.at[0]], sem)
    c2 = pltpu.async_copy(xrows_v, xd_hbm.at[dst_v.at[1]], sem)
    c1.wait()
    c2.wait()

    @pl.when(wid == 0)
    def _bexp():
        bstart = lax.shift_right_logical(pad_off, 8)
        nb = lax.shift_right_logical(pc, 8)
        nblk = jnp.sum(nb)
        elast = jnp.max(jnp.where(totals > 0, lanes, zeros))
        bs = [jnp.sum(jnp.where(lanes == e, bstart, zeros))
              for e in range(N_EXP)]
        nbs = [jnp.sum(jnp.where(lanes == e, nb, zeros))
               for e in range(N_EXP)]
        for v in range(2):
            bi = lanes + v * 16
            be = jnp.full((16,), 0, jnp.int32) + elast
            for e in range(N_EXP):
                msk = jnp.logical_and(bi >= bs[e], bi < bs[e] + nbs[e])
                be = jnp.where(msk, e, be)
            be = jnp.where(bi == NB, nblk, be)
            bexp_v[pl.ds(v * 16, 16)] = be
        pltpu.sync_copy(bexp_v, bexp_hbm)


def _gelu(h):
    return 0.5 * h * (1.0 + lax.erf(h * 0.7071067811865476))


def _grouped_body(s_ref, xd_ref, w1a_ref, w1b_ref, b1_ref, w2a_ref, w2b_ref,
                  b2_ref, yd_ref):
    b = pl.program_id(0)
    e = s_ref[b]
    nblk = s_ref[NB]

    @pl.when(b < nblk)
    def _compute():
        x = xd_ref[...]
        ha = lax.dot_general(
            x, w1a_ref[0], (((1,), (1,)), ((), ())),
            preferred_element_type=jnp.float32)
        hb = lax.dot_general(
            x, w1b_ref[0], (((1,), (1,)), ((), ())),
            preferred_element_type=jnp.float32)
        b1_row = b1_ref[pl.ds(e, 1), :]
        g = _gelu(jnp.concatenate([ha, hb], axis=1) + b1_row)
        pa = lax.dot_general(
            g, w2a_ref[0], (((1,), (1,)), ((), ())),
            preferred_element_type=jnp.float32)
        pb = lax.dot_general(
            g, w2b_ref[0], (((1,), (1,)), ((), ())),
            preferred_element_type=jnp.float32)
        yd_ref[...] = (jnp.concatenate([pa, pb], axis=1)
                       + b2_ref[pl.ds(e, 1), :])


_QB = 16
_NQ = TPW // _QB


def _combine_body(yd_hbm, pos_hbm, wasgn_hbm, out_hbm,
                  p1_v, p2_v, w1_v, w2_v, rows1_v, rows2_v, ob_v,
                  sem_a, sem_b):
    wid = lax.axis_index("s") * NC + lax.axis_index("c")
    base = wid * TPW

    pltpu.sync_copy(wasgn_hbm.at[0, pl.ds(base, TPW)], w1_v)
    pltpu.sync_copy(wasgn_hbm.at[1, pl.ds(base, TPW)], w2_v)
    pltpu.sync_copy(pos_hbm.at[0, pl.ds(base, TPW)], p1_v)
    pltpu.sync_copy(pos_hbm.at[1, pl.ds(base, TPW)], p2_v)

    sems = [sem_a, sem_b]

    def _issue(q):
        s = sems[q % 2]
        c1 = pltpu.async_copy(yd_hbm.at[p1_v.at[pl.ds(q * _QB, _QB)]],
                              rows1_v.at[q % 2], s)
        c2 = pltpu.async_copy(yd_hbm.at[p2_v.at[pl.ds(q * _QB, _QB)]],
                              rows2_v.at[q % 2], s)
        return c1, c2

    pend = _issue(0)
    for q in range(_NQ):
        pend[0].wait()
        pend[1].wait()
        if q + 1 < _NQ:
            pend = _issue(q + 1)
        qq = q % 2

        def _row(r, carry):
            idx = jnp.full((16,), q * _QB + r, jnp.int32)
            w1s = plsc.load_gather(w1_v, [idx])
            w2s = plsc.load_gather(w2_v, [idx])

            def _col(c, carry2):
                for u in range(4):
                    a = rows1_v[qq, r, pl.ds((c * 4 + u) * 16, 16)]
                    bb = rows2_v[qq, r, pl.ds((c * 4 + u) * 16, 16)]
                    ob_v[r, pl.ds((c * 4 + u) * 16, 16)] = w1s * a + w2s * bb
                return carry2

            lax.fori_loop(0, D // 64, _col, 0)
            return carry

        lax.fori_loop(0, _QB, _row, 0)
        pltpu.sync_copy(ob_v, out_hbm.at[pl.ds(base + q * _QB, _QB)])


_SC_MESH = plsc.VectorSubcoreMesh(core_axis_name="c", subcore_axis_name="s",
                                  num_cores=NC, num_subcores=NS)

_dispatch = pl.kernel(
    _dispatch_body,
    out_type=(
        jax.ShapeDtypeStruct((P, D), jnp.float32),
        jax.ShapeDtypeStruct((2, N_TOK), jnp.int32),
        jax.ShapeDtypeStruct((NB + 8, ), jnp.int32),
    ),
    mesh=_SC_MESH,
    compiler_params=pltpu.CompilerParams(needs_layout_passes=False),
    scratch_types=[
        pltpu.VMEM((NW, 16), jnp.int32),
        pltpu.VMEM((2, TPW), jnp.int32),
        pltpu.VMEM((2, TPW), jnp.int32),
        pltpu.VMEM((TPW, D), jnp.float32),
        pltpu.VMEM((NB + 8,), jnp.int32),
        pltpu.SemaphoreType.DMA,
    ],
)

_combine = pl.kernel(
    _combine_body,
    out_type=jax.ShapeDtypeStruct((N_TOK, D), jnp.float32),
    mesh=_SC_MESH,
    compiler_params=pltpu.CompilerParams(needs_layout_passes=False),
    scratch_types=[
        pltpu.VMEM((TPW,), jnp.int32),
        pltpu.VMEM((TPW,), jnp.int32),
        pltpu.VMEM((TPW,), jnp.float32),
        pltpu.VMEM((TPW,), jnp.float32),
        pltpu.VMEM((2, _QB, D), jnp.float32),
        pltpu.VMEM((2, _QB, D), jnp.float32),
        pltpu.VMEM((_QB, D), jnp.float32),
        pltpu.SemaphoreType.DMA,
        pltpu.SemaphoreType.DMA,
    ],
)


@jax.jit
def kernel(x, W_gate, W1, b1, W2, b2):
    x_flat = x.reshape(N_TOK, D)

    easgn, wasgn, counts, aux = pl.pallas_call(
        _router_body,
        out_shape=(
            jax.ShapeDtypeStruct((2, N_TOK), jnp.int32),
            jax.ShapeDtypeStruct((2, N_TOK), jnp.float32),
            jax.ShapeDtypeStruct((NW, 16), jnp.int32),
            jax.ShapeDtypeStruct((1, 1), jnp.float32),
        ),
    )(x_flat, W_gate)

    xd, pos, bexp = _dispatch(x_flat, easgn, counts)

    yd = pl.pallas_call(
        _grouped_body,
        grid_spec=pltpu.PrefetchScalarGridSpec(
            num_scalar_prefetch=1,
            grid=(NB,),
            in_specs=[
                pl.BlockSpec((TB, D), lambda b, s: (b, 0)),
                pl.BlockSpec((1, DFF // 2, D), lambda b, s: (s[b], 0, 0)),
                pl.BlockSpec((1, DFF // 2, D), lambda b, s: (s[b], 1, 0)),
                pl.BlockSpec((N_EXP, DFF), lambda b, s: (0, 0)),
                pl.BlockSpec((1, D // 2, DFF), lambda b, s: (s[b], 0, 0),
                             pipeline_mode=pl.Buffered(buffer_count=1)),
                pl.BlockSpec((1, D // 2, DFF), lambda b, s: (s[b], 1, 0),
                             pipeline_mode=pl.Buffered(buffer_count=1)),
                pl.BlockSpec((N_EXP, D), lambda b, s: (0, 0)),
            ],
            out_specs=pl.BlockSpec((TB, D), lambda b, s: (b, 0)),
        ),
        out_shape=jax.ShapeDtypeStruct((P, D), jnp.float32),
        compiler_params=pltpu.CompilerParams(
            dimension_semantics=("arbitrary",),
            vmem_limit_bytes=60 * 1024 * 1024,
        ),
    )(bexp, xd, W1, W1, b1, W2, W2, b2)

    out = _combine(yd, pos, wasgn)
    return out.reshape(1, N_TOK, D), aux.reshape(())
